# baseline (speedup 1.0000x reference)
import functools
import jax
import jax.numpy as jnp
from jax import lax
from jax.experimental import pallas as pl
from jax.experimental.pallas import tpu as pltpu
from jax.experimental.pallas import tpu_sc as plsc

DIM = 32
B = 4096
T = 200
NC, NS = 2, 16
NW = NC * NS
PB = B // NW
TB = 8
NCHUNK = T // TB

_mesh = plsc.VectorSubcoreMesh(core_axis_name="c", subcore_axis_name="s")


@functools.partial(
    pl.kernel,
    mesh=_mesh,
    compiler_params=pltpu.CompilerParams(
        use_tc_tiling_on_sc=False, needs_layout_passes=False
    ),
    out_type=jax.ShapeDtypeStruct((T, NW * 4, 8, 128), jnp.float32),
    scratch_types=[
        pltpu.VMEM((PB, TB), jnp.int32),
        pltpu.VMEM((TB, PB), jnp.int32),
        pltpu.VMEM((TB, PB), jnp.int32),
        pltpu.VMEM((TB * PB, DIM), jnp.float32),
        pltpu.VMEM((TB * PB, DIM), jnp.float32),
        pltpu.VMEM((TB * 4, 8, 128), jnp.float32),
        pltpu.SemaphoreType.DMA,
        pltpu.SemaphoreType.DMA,
        pltpu.SemaphoreType.DMA,
    ],
)
def _embed(ids_hbm, table_hbm, out_hbm, ist, ix0, ix1, g0, g1, tr,
           sg0, sg1, sw):
    wid = lax.axis_index("s") * NC + lax.axis_index("c")
    rbase = wid * PB
    ixs = (ix0, ix1)
    gs = (g0, g1)
    sg = (sg0, sg1)
    iota = lax.iota(jnp.int32, 16)

    def load_ids(g):
        pltpu.sync_copy(
            ids_hbm.at[pl.ds(rbase, PB), pl.ds(g * TB, TB)], ist
        )

    def transpose_ids(b):

        @plsc.parallel_loop(0, TB * 8, unroll=4)
        def _(i):
            tt = i >> 3
            g16 = i & 7
            rows = iota + (g16 * 16)
            cols = jnp.zeros((16,), jnp.int32) + tt
            v = plsc.load_gather(ist, [rows, cols])
            ixs[b][tt, pl.ds(g16 * 16, 16)] = v

    def fire(b):
        for tt in range(TB):
            pltpu.async_copy(
                table_hbm.at[ixs[b].at[tt]],
                gs[b].at[pl.ds(tt * PB, PB)],
                sg[b],
            )

    def wait_gather(b):
        pltpu.make_async_copy(
            table_hbm.at[pl.ds(0, TB * PB)], gs[b], sg[b]
        ).wait()

    def transpose_rows(b):

        @plsc.parallel_loop(0, TB * 32, unroll=16)
        def _(i):
            tt = i >> 5
            d = i & 31
            base = tt * PB
            cols = jnp.zeros((16,), jnp.int32) + d
            for g16 in range(8):
                rows = iota + (base + g16 * 16)
                v = plsc.load_gather(gs[b], [rows, cols])
                tr[tt * 4 + (d >> 3), d & 7, pl.ds(g16 * 16, 16)] = v

    def fire_write(g):
        for tt in range(TB):
            for dt in range(4):
                pltpu.async_copy(
                    tr.at[tt * 4 + dt],
                    out_hbm.at[g * TB + tt, dt * NW + wid],
                    sw,
                )

    def wait_write():
        pltpu.make_async_copy(
            out_hbm.at[0, pl.ds(0, TB * 4)], tr, sw
        ).wait()

    load_ids(0)
    transpose_ids(0)
    fire(0)
    load_ids(1)
    transpose_ids(1)
    fire(1)

    def body(i, _):
        for par in range(2):
            g = 2 * i + par

            @pl.when(g < NCHUNK)
            def _():
                wait_gather(par)

                @pl.when(g >= 1)
                def _():
                    wait_write()

                transpose_rows(par)
                fire_write(g)

                @pl.when(g + 2 < NCHUNK)
                def _():
                    load_ids(g + 2)
                    transpose_ids(par)
                    fire(par)

        return 0

    lax.fori_loop(0, (NCHUNK + 2) // 2, body, 0)
    wait_write()


def kernel(ids, table):
    p = _embed(ids, table)
    return (
        p.reshape(T, 4, NW, 8, 128)
        .transpose(2, 4, 0, 1, 3)
        .reshape(B, T, DIM)
    )


# speedup vs baseline: 1.0220x; 1.0220x over previous
import functools
import jax
import jax.numpy as jnp
from jax import lax
from jax.experimental import pallas as pl
from jax.experimental.pallas import tpu as pltpu
from jax.experimental.pallas import tpu_sc as plsc

DIM = 32
B = 4096
T = 200
NC, NS = 2, 16
NW = NC * NS
PB = B // NW
TB = 8
NCHUNK = T // TB

_mesh = plsc.VectorSubcoreMesh(core_axis_name="c", subcore_axis_name="s")


@functools.partial(
    pl.kernel,
    mesh=_mesh,
    compiler_params=pltpu.CompilerParams(
        use_tc_tiling_on_sc=False, needs_layout_passes=False
    ),
    out_type=jax.ShapeDtypeStruct((T, NW * 4, 8, 128), jnp.float32),
    scratch_types=[
        pltpu.VMEM((PB, TB), jnp.int32),
        pltpu.VMEM((TB, PB), jnp.int32),
        pltpu.VMEM((TB, PB), jnp.int32),
        pltpu.VMEM((TB * PB, DIM), jnp.float32),
        pltpu.VMEM((TB * PB, DIM), jnp.float32),
        pltpu.VMEM((TB * 4, 8, 128), jnp.float32),
        pltpu.SemaphoreType.DMA,
        pltpu.SemaphoreType.DMA,
        pltpu.SemaphoreType.DMA,
    ],
)
def _embed(ids_hbm, table_hbm, out_hbm, ist, ix0, ix1, g0, g1, tr,
           sg0, sg1, sw):
    wid = lax.axis_index("s") * NC + lax.axis_index("c")
    rbase = wid * PB
    ixs = (ix0, ix1)
    gs = (g0, g1)
    sg = (sg0, sg1)
    iota = lax.iota(jnp.int32, 16)

    def load_ids(g):
        pltpu.sync_copy(
            ids_hbm.at[pl.ds(rbase, PB), pl.ds(g * TB, TB)], ist
        )

    def transpose_ids(b):

        @plsc.parallel_loop(0, TB * 8, unroll=4)
        def _(i):
            tt = i >> 3
            g16 = i & 7
            rows = iota + (g16 * 16)
            cols = jnp.zeros((16,), jnp.int32) + tt
            v = plsc.load_gather(ist, [rows, cols])
            ixs[b][tt, pl.ds(g16 * 16, 16)] = v

    def fire(b):
        for tt in range(TB):
            pltpu.async_copy(
                table_hbm.at[ixs[b].at[tt]],
                gs[b].at[pl.ds(tt * PB, PB)],
                sg[b],
            )

    def wait_gather(b):
        pltpu.make_async_copy(
            table_hbm.at[pl.ds(0, TB * PB)], gs[b], sg[b]
        ).wait()

    def transpose_rows(b):

        @plsc.parallel_loop(0, TB * 32, unroll=8)
        def _(i):
            tt = i >> 5
            d = i & 31
            base = tt * PB
            cols = jnp.zeros((16,), jnp.int32) + d
            for g16 in range(8):
                rows = iota + (base + g16 * 16)
                v = plsc.load_gather(gs[b], [rows, cols])
                tr[tt * 4 + (d >> 3), d & 7, pl.ds(g16 * 16, 16)] = v

    def fire_write(g):
        for tt in range(TB):
            for dt in range(4):
                pltpu.async_copy(
                    tr.at[tt * 4 + dt],
                    out_hbm.at[g * TB + tt, dt * NW + wid],
                    sw,
                )

    def wait_write():
        pltpu.make_async_copy(
            out_hbm.at[0, pl.ds(0, TB * 4)], tr, sw
        ).wait()

    load_ids(0)
    transpose_ids(0)
    fire(0)
    load_ids(1)
    transpose_ids(1)
    fire(1)

    def body(i, _):
        for par in range(2):
            g = 2 * i + par

            @pl.when(g < NCHUNK)
            def _():
                wait_gather(par)

                @pl.when(g >= 1)
                def _():
                    wait_write()

                transpose_rows(par)
                fire_write(g)

                @pl.when(g + 2 < NCHUNK)
                def _():
                    load_ids(g + 2)
                    transpose_ids(par)
                    fire(par)

        return 0

    lax.fori_loop(0, (NCHUNK + 2) // 2, body, 0)
    wait_write()


def kernel(ids, table):
    p = _embed(ids, table)
    return (
        p.reshape(T, 4, NW, 8, 128)
        .transpose(2, 4, 0, 1, 3)
        .reshape(B, T, DIM)
    )
